# baseline (speedup 1.0000x reference)
import math

import jax
import jax.numpy as jnp
from jax.experimental import pallas as pl

N_NODES = 10000
N_EDGES = 320000
D = 128
DEPTH = 3


def _relu_block(x_ref, o_ref):
    o_ref[...] = jnp.maximum(x_ref[...], 0.0)


def _pallas_relu(x):
    n, d = x.shape
    blk = 512
    npad = ((n + blk - 1) // blk) * blk
    xp = jnp.pad(x, ((0, npad - n), (0, 0)))
    out = pl.pallas_call(
        _relu_block,
        grid=(npad // blk,),
        in_specs=[pl.BlockSpec((blk, d), lambda i: (i, 0))],
        out_specs=pl.BlockSpec((blk, d), lambda i: (i, 0)),
        out_shape=jax.ShapeDtypeStruct((npad, d), x.dtype),
    )(xp)
    return out[:n]


def _gcn_conv(x, edge_index, emask, W, b):
    N = x.shape[0]
    loop = jnp.arange(N, dtype=edge_index.dtype)
    src = jnp.concatenate([edge_index[0], loop])
    dst = jnp.concatenate([edge_index[1], loop])
    m = jnp.concatenate([emask.astype(x.dtype), jnp.ones((N,), dtype=x.dtype)])
    deg = jnp.zeros((N,), dtype=x.dtype).at[dst].add(m)
    dis = jnp.where(deg > 0, 1.0 / jnp.sqrt(deg), 0.0)
    norm = dis[src] * dis[dst] * m
    xw = x @ W
    out = jnp.zeros((N, W.shape[1]), dtype=x.dtype).at[dst].add(xw[src] * norm[:, None])
    return out + b


def _graph_conv_score(x, edge_index, emask, Wrel, brel, Wroot):
    m = emask.astype(x.dtype)
    agg = jnp.zeros_like(x).at[edge_index[1]].add(x[edge_index[0]] * m[:, None])
    return (agg @ Wrel + brel) + x @ Wroot


def _sag_pool(x, edge_index, emask, batch, p):
    score = jnp.tanh(_graph_conv_score(x, edge_index, emask, p['Wrel'], p['brel'], p['Wroot'])[:, 0])
    N = x.shape[0]
    k = int(math.ceil(0.5 * N))
    perm = jnp.argsort(-score)[:k]
    x_p = x[perm] * score[perm][:, None]
    batch_p = batch[perm]
    mask = jnp.full((N,), -1, dtype=jnp.int32).at[perm].set(jnp.arange(k, dtype=jnp.int32))
    row = mask[edge_index[0]]
    col = mask[edge_index[1]]
    valid = emask & (row >= 0) & (col >= 0)
    new_ei = jnp.stack([jnp.where(valid, row, 0), jnp.where(valid, col, 0)]).astype(edge_index.dtype)
    return x_p, new_ei, valid, batch_p, perm, score


def _unpool_edges(edge_index_cg, perm):
    rev = jnp.argsort(perm)
    return jnp.stack([rev[edge_index_cg[0]], rev[edge_index_cg[1]]]).astype(edge_index_cg.dtype)


def kernel(x, edge_index, params):
    batch = jnp.zeros((x.shape[0],), dtype=jnp.int32)
    emask = jnp.ones((edge_index.shape[1],), dtype=jnp.bool_)
    skips = []
    for i in range(DEPTH):
        x = _pallas_relu(_gcn_conv(x, edge_index, emask, params['down_W'][i], params['down_b'][i]))
        skips.append(x)
        if i != DEPTH - 1:
            x, edge_index, emask, batch, _, _ = _sag_pool(x, edge_index, emask, batch, params['pool'][i])
    for i in range(DEPTH - 1):
        x_cg, ei_cg, emask_cg, batch_cg, perm, _ = _sag_pool(x, edge_index, emask, batch, params['unpool'][i])
        new_batch = batch[perm]
        x_un = x[perm]
        ei_un = _unpool_edges(ei_cg, perm)
        x, edge_index, emask, batch = x_un, ei_un, emask_cg, new_batch
        skip = skips[-(i + 1)]
        if x.shape[0] != skip.shape[0]:
            skip = skip[:x.shape[0]]
        x = jnp.concatenate([x, skip], axis=1)
        x = _pallas_relu(_gcn_conv(x, edge_index, emask, params['up_W'][i], params['up_b'][i]))
    return x


# speedup vs baseline: 1.0100x; 1.0100x over previous
import math

import jax
import jax.numpy as jnp
from jax.experimental import pallas as pl

N_NODES = 10000
N_EDGES = 320000
D = 128
DEPTH = 3


def _bias_relu_block(x_ref, b_ref, o_ref):
    o_ref[...] = jnp.maximum(x_ref[...] + b_ref[...], 0.0)


def _pallas_bias_relu(x, b):
    n, d = x.shape
    blk = 512
    npad = ((n + blk - 1) // blk) * blk
    xp = jnp.pad(x, ((0, npad - n), (0, 0)))
    out = pl.pallas_call(
        _bias_relu_block,
        grid=(npad // blk,),
        in_specs=[pl.BlockSpec((blk, d), lambda i: (i, 0)),
                  pl.BlockSpec((1, d), lambda i: (0, 0))],
        out_specs=pl.BlockSpec((blk, d), lambda i: (i, 0)),
        out_shape=jax.ShapeDtypeStruct((npad, d), x.dtype),
    )(xp, b.reshape(1, d))
    return out[:n]


def _mm_block(x_ref, w_ref, o_ref):
    o_ref[...] = jnp.dot(x_ref[...], w_ref[...])


def _pallas_matmul(x, W):
    n, f = x.shape
    fo = W.shape[1]
    blk = 512
    npad = ((n + blk - 1) // blk) * blk
    xp = jnp.pad(x, ((0, npad - n), (0, 0)))
    out = pl.pallas_call(
        _mm_block,
        grid=(npad // blk,),
        in_specs=[pl.BlockSpec((blk, f), lambda i: (i, 0)),
                  pl.BlockSpec((f, fo), lambda i: (0, 0))],
        out_specs=pl.BlockSpec((blk, fo), lambda i: (i, 0)),
        out_shape=jax.ShapeDtypeStruct((npad, fo), x.dtype),
    )(xp, W)
    return out[:n]


def _gcn_conv_pre_bias(x, edge_index, emask, W):
    N = x.shape[0]
    loop = jnp.arange(N, dtype=edge_index.dtype)
    src = jnp.concatenate([edge_index[0], loop])
    dst = jnp.concatenate([edge_index[1], loop])
    m = jnp.concatenate([emask.astype(x.dtype), jnp.ones((N,), dtype=x.dtype)])
    deg = jnp.zeros((N,), dtype=x.dtype).at[dst].add(m)
    dis = jnp.where(deg > 0, 1.0 / jnp.sqrt(deg), 0.0)
    norm = dis[src] * dis[dst] * m
    xw = _pallas_matmul(x, W)
    out = jnp.zeros((N, W.shape[1]), dtype=x.dtype).at[dst].add(xw[src] * norm[:, None])
    return out


def _graph_conv_score(x, edge_index, emask, Wrel, brel, Wroot):
    m = emask.astype(x.dtype)
    agg = jnp.zeros_like(x).at[edge_index[1]].add(x[edge_index[0]] * m[:, None])
    return (agg @ Wrel + brel) + x @ Wroot


def _sag_pool(x, edge_index, emask, batch, p):
    score = jnp.tanh(_graph_conv_score(x, edge_index, emask, p['Wrel'], p['brel'], p['Wroot'])[:, 0])
    N = x.shape[0]
    k = int(math.ceil(0.5 * N))
    perm = jnp.argsort(-score)[:k]
    x_p = x[perm] * score[perm][:, None]
    batch_p = batch[perm]
    mask = jnp.full((N,), -1, dtype=jnp.int32).at[perm].set(jnp.arange(k, dtype=jnp.int32))
    row = mask[edge_index[0]]
    col = mask[edge_index[1]]
    valid = emask & (row >= 0) & (col >= 0)
    new_ei = jnp.stack([jnp.where(valid, row, 0), jnp.where(valid, col, 0)]).astype(edge_index.dtype)
    return x_p, new_ei, valid, batch_p, perm, score


def _unpool_edges(edge_index_cg, perm):
    rev = jnp.argsort(perm)
    return jnp.stack([rev[edge_index_cg[0]], rev[edge_index_cg[1]]]).astype(edge_index_cg.dtype)


def kernel(x, edge_index, params):
    batch = jnp.zeros((x.shape[0],), dtype=jnp.int32)
    emask = jnp.ones((edge_index.shape[1],), dtype=jnp.bool_)
    skips = []
    for i in range(DEPTH):
        x = _pallas_bias_relu(
            _gcn_conv_pre_bias(x, edge_index, emask, params['down_W'][i]),
            params['down_b'][i])
        skips.append(x)
        if i != DEPTH - 1:
            x, edge_index, emask, batch, _, _ = _sag_pool(x, edge_index, emask, batch, params['pool'][i])
    for i in range(DEPTH - 1):
        x_cg, ei_cg, emask_cg, batch_cg, perm, _ = _sag_pool(x, edge_index, emask, batch, params['unpool'][i])
        new_batch = batch[perm]
        x_un = x[perm]
        ei_un = _unpool_edges(ei_cg, perm)
        x, edge_index, emask, batch = x_un, ei_un, emask_cg, new_batch
        skip = skips[-(i + 1)]
        if x.shape[0] != skip.shape[0]:
            skip = skip[:x.shape[0]]
        x = jnp.concatenate([x, skip], axis=1)
        x = _pallas_bias_relu(
            _gcn_conv_pre_bias(x, edge_index, emask, params['up_W'][i]),
            params['up_b'][i])
    return x
